# baseline (speedup 1.0000x reference)
import functools

import jax
import jax.numpy as jnp
from jax import lax
from jax.experimental import pallas as pl
from jax.experimental.pallas import tpu as pltpu
from jax.experimental.pallas import tpu_sc as plsc

N_NODES = 10000
N_EDGES = 320000
NS = 16
NW = 32
CH = 128
NCH = 80
EPW = CH * NCH
E_PAD = NW * EPW
NCHT = N_EDGES // CH
NCHR = NCHT - (NW - 1) * NCH
NBC = 5
NG = NCH // NBC
NP = 10240
RPT = NP // NS

_MESH = plsc.VectorSubcoreMesh(core_axis_name="c", subcore_axis_name="s")
_SC_PARAMS = pltpu.CompilerParams(use_tc_tiling_on_sc=False)


def _make_agg(F):
    @functools.partial(
        pl.kernel,
        out_type=jax.ShapeDtypeStruct((2, NP, F), jnp.float32),
        mesh=_MESH,
        compiler_params=_SC_PARAMS,
        scratch_types=[
            pltpu.VMEM((NCH, CH), jnp.int32),
            pltpu.VMEM((NCH, CH), jnp.int32),
            pltpu.VMEM((NBC, CH, F), jnp.float32),
            pltpu.VMEM((NBC, CH, F), jnp.float32),
            pltpu.VMEM_SHARED((NP, F), jnp.float32),
            pltpu.SemaphoreType.DMA,
            pltpu.SemaphoreType.DMA,
            pltpu.SemaphoreType.DMA,
            pltpu.SemaphoreType.DMA,
        ],
    )
    def agg(g_hbm, ei_hbm, pad_hbm, z_hbm, out_hbm, src_v, dst_v, bufa, bufb,
            acc_sh, gsema, gsemb, ssema, ssemb):
        c = lax.axis_index("c")
        s = lax.axis_index("s")
        w = c * NS + s
        pltpu.sync_copy(z_hbm.at[pl.ds(s * RPT, RPT)],
                        acc_sh.at[pl.ds(s * RPT, RPT)])

        @pl.when(w < NW - 1)
        def _():
            pltpu.sync_copy(ei_hbm.at[0, pl.ds(w * NCH, NCH)], src_v)
            pltpu.sync_copy(ei_hbm.at[1, pl.ds(w * NCH, NCH)], dst_v)

        @pl.when(w == NW - 1)
        def _():
            pltpu.sync_copy(ei_hbm.at[0, pl.ds(w * NCH, NCHR)],
                            src_v.at[pl.ds(0, NCHR)])
            pltpu.sync_copy(ei_hbm.at[1, pl.ds(w * NCH, NCHR)],
                            dst_v.at[pl.ds(0, NCHR)])
            pltpu.sync_copy(pad_hbm.at[0], src_v.at[pl.ds(NCHR, NCH - NCHR)])
            pltpu.sync_copy(pad_hbm.at[1], dst_v.at[pl.ds(NCHR, NCH - NCHR)])

        plsc.subcore_barrier()

        def gather(ch, buf, b, sem):
            pltpu.async_copy(g_hbm.at[src_v.at[ch]], buf.at[b], sem)

        def gather_wait(buf, b, sem):
            pltpu.make_async_copy(g_hbm.at[src_v.at[0]], buf.at[b], sem).wait()

        def scat(ch, buf, b, sem):
            pltpu.async_copy(buf.at[b], acc_sh.at[dst_v.at[ch]], sem, add=True)

        def scat_wait(buf, b, sem):
            pltpu.make_async_copy(buf.at[b], acc_sh.at[dst_v.at[0]],
                                  sem).wait()

        for b in range(NBC):
            gather(b, bufa, b, gsema)

        def phase(j, buf, gsem, obuf, osem_s, gsem_o, first, last):
            for b in range(NBC):
                gather_wait(buf, b, gsem)
            for b in range(NBC):
                scat(j * NBC + b, buf, b,
                     ssema if buf is bufa else ssemb)
            if first is None:
                for b in range(NBC):
                    scat_wait(obuf, b, osem_s)
            else:
                @pl.when(jnp.logical_not(first))
                def _():
                    for b in range(NBC):
                        scat_wait(obuf, b, osem_s)

            @pl.when(jnp.logical_not(last))
            def _():
                for b in range(NBC):
                    gather((j + 1) * NBC + b, obuf, b, gsem_o)

        def body(j, carry):
            @pl.when(j % 2 == 0)
            def _():
                phase(j, bufa, gsema, bufb, ssemb, gsemb,
                      first=j == 0, last=j >= NG - 1)

            @pl.when(j % 2 == 1)
            def _():
                phase(j, bufb, gsemb, bufa, ssema, gsema,
                      first=None, last=j >= NG - 1)

            return carry

        lax.fori_loop(0, NG, body, 0)
        for b in range(NBC):
            scat_wait(bufb if NG % 2 == 0 else bufa, b,
                      ssemb if NG % 2 == 0 else ssema)
        plsc.subcore_barrier()
        pltpu.sync_copy(acc_sh.at[pl.ds(s * RPT, RPT)],
                        out_hbm.at[c, pl.ds(s * RPT, RPT)])

    return agg


_agg16 = _make_agg(16)


@functools.partial(
    pl.kernel,
    out_type=jax.ShapeDtypeStruct((2, NP, 16), jnp.float32),
    mesh=_MESH,
    compiler_params=_SC_PARAMS,
    scratch_types=[
        pltpu.VMEM((NCH, CH), jnp.int32),
        pltpu.VMEM((CH, 16), jnp.float32),
        pltpu.VMEM_SHARED((NP, 16), jnp.float32),
        pltpu.SemaphoreType.DMA,
    ],
)
def _deg(ei_hbm, pad_hbm, ones_hbm, z_hbm, out_hbm, dst_v, ones_v, acc_sh,
         sem):
    c = lax.axis_index("c")
    s = lax.axis_index("s")
    w = c * NS + s
    pltpu.sync_copy(z_hbm.at[pl.ds(s * RPT, RPT)],
                    acc_sh.at[pl.ds(s * RPT, RPT)])
    pltpu.sync_copy(ones_hbm, ones_v)

    @pl.when(w < NW - 1)
    def _():
        pltpu.sync_copy(ei_hbm.at[1, pl.ds(w * NCH, NCH)], dst_v)

    @pl.when(w == NW - 1)
    def _():
        pltpu.sync_copy(ei_hbm.at[1, pl.ds(w * NCH, NCHR)],
                        dst_v.at[pl.ds(0, NCHR)])
        pltpu.sync_copy(pad_hbm.at[1], dst_v.at[pl.ds(NCHR, NCH - NCHR)])

    plsc.subcore_barrier()

    K = 8

    def scat(ch):
        pltpu.async_copy(ones_v, acc_sh.at[dst_v.at[ch]], sem, add=True)

    def scat_wait():
        pltpu.make_async_copy(ones_v, acc_sh.at[dst_v.at[0]], sem).wait()

    for b in range(K):
        scat(b)

    def body(j, carry):
        for b in range(K):
            scat(j * K + b)
        for b in range(K):
            scat_wait()
        return carry

    lax.fori_loop(1, NCH // K, body, 0)
    for b in range(K):
        scat_wait()
    plsc.subcore_barrier()
    pltpu.sync_copy(acc_sh.at[pl.ds(s * RPT, RPT)],
                    out_hbm.at[c, pl.ds(s * RPT, RPT)])


NPF = NP * 16 // 128
NF1 = N_NODES * 16 // 128


def _k1a_body(x3_ref, we_ref, h_ref):
    hf = jnp.dot(x3_ref[:, 0, :], we_ref[0],
                 preferred_element_type=jnp.float32)
    for a in range(1, 8):
        hf = hf + jnp.dot(x3_ref[:, a, :], we_ref[a],
                          preferred_element_type=jnp.float32)
    h_ref[...] = hf


def _k1b_body(hf_ref, daccf_ref, g_ref, dis_ref):
    degf = daccf_ref[0] + daccf_ref[1] + 1.0
    disf = lax.rsqrt(degf)
    dis_ref[...] = disf
    g = hf_ref[...] * disf[:NF1]
    g_ref[...] = jnp.concatenate(
        [g, jnp.zeros((NPF - NF1, 128), jnp.float32)], axis=0)


def _k2_body(af_ref, gf_ref, disf_ref, bf_ref, wf_ref, out_ref):
    disf = disf_ref[...]
    z = jnp.maximum(disf * (af_ref[0] + af_ref[1] + gf_ref[...]) + bf_ref[...],
                    0.0)
    out_ref[...] = (
        jnp.dot(z, wf_ref[...], preferred_element_type=jnp.float32) * disf)


def _k3_body(af_ref, gf_ref, disf_ref, bf_ref, out_ref):
    disf = disf_ref[...]
    z = jnp.maximum(disf * (af_ref[0] + af_ref[1] + gf_ref[...]) + bf_ref[...],
                    0.0)
    out_ref[...] = z * disf


def _k4_body(af_ref, gf_ref, disf_ref, we_ref, b_ref, out_ref):
    qdf = disf_ref[...] * (af_ref[0] + af_ref[1] + gf_ref[...])
    hcat = jnp.dot(qdf, we_ref[...], preferred_element_type=jnp.float32)
    cols = []
    for a in range(8):
        p = hcat[:, 40 * a:40 * (a + 1)] + b_ref[...]
        m = jnp.max(p, axis=1, keepdims=True)
        e = jnp.exp(p - m)
        o = p - m - jnp.log(jnp.sum(e, axis=1, keepdims=True))
        cols.append(jnp.reshape(o, (NPF, 1, 40)))
    full = jnp.concatenate(cols, axis=1).reshape(NP, 40)
    out_ref[...] = full[:N_NODES]


def _f32(*shape):
    return jax.ShapeDtypeStruct(shape, jnp.float32)


_K1A = pl.pallas_call(_k1a_body, out_shape=_f32(NF1, 128))
_K1B = pl.pallas_call(_k1b_body, out_shape=[_f32(NPF, 128), _f32(NPF, 128)])
_K2 = pl.pallas_call(_k2_body, out_shape=_f32(NPF, 128))
_K3 = pl.pallas_call(_k3_body, out_shape=_f32(NPF, 128))
_K4 = pl.pallas_call(_k4_body, out_shape=_f32(N_NODES, 40))


def kernel(x, edge_index, W1, b1, W2, b2, W3, b3):
    npad = E_PAD - N_EDGES
    k = jnp.arange(npad, dtype=jnp.int32)
    pad3 = jnp.stack([k % N_NODES, N_NODES + k % (NP - N_NODES)]
                     ).reshape(2, NCH - NCHR, CH)
    ei = edge_index.reshape(2, NCHT, CH)
    z16 = jnp.zeros((NP, 16), jnp.float32)
    ones = jnp.ones((CH, 16), jnp.float32)
    e8 = jnp.eye(8, dtype=jnp.float32)
    w1e = (e8[:, None, :, None] * W1[None, :, None, :]).reshape(8, 128, 128)
    w2f = (e8[:, None, :, None] * W2[None, :, None, :]).reshape(128, 128)

    hf = _K1A(x.reshape(NF1, 8, 128), w1e)
    dacc = _deg(ei, pad3, ones, z16)
    g1f, disf = _K1B(hf, dacc.reshape(2, NPF, 128))
    a1 = _agg16(g1f.reshape(NP, 16), ei, pad3, z16)
    g2f = _K2(a1.reshape(2, NPF, 128), g1f, disf,
              jnp.tile(b1, 8).reshape(1, 128), w2f)
    a2 = _agg16(g2f.reshape(NP, 16), ei, pad3, z16)
    g3f = _K3(a2.reshape(2, NPF, 128), g2f, disf,
              jnp.tile(b2, 8).reshape(1, 128))
    a3 = _agg16(g3f.reshape(NP, 16), ei, pad3, z16)
    w3e = jnp.transpose(
        (e8[:, :, None, None] * W3[None, None, :, :]).reshape(8, 128, 40),
        (1, 0, 2)).reshape(128, 320)
    return _K4(a3.reshape(2, NPF, 128), g3f, disf, w3e, b3.reshape(1, 40))


# speedup vs baseline: 1.0718x; 1.0718x over previous
import functools

import jax
import jax.numpy as jnp
from jax import lax
from jax.experimental import pallas as pl
from jax.experimental.pallas import tpu as pltpu
from jax.experimental.pallas import tpu_sc as plsc

N_NODES = 10000
N_EDGES = 320000
NS = 16
NW = 32
CH = 128
NCH = 80
EPW = CH * NCH
E_PAD = NW * EPW
NCHT = N_EDGES // CH
NCHR = NCHT - (NW - 1) * NCH
NBC = 5
NG = NCH // NBC
NP = 10240
RPT = NP // NS

_MESH = plsc.VectorSubcoreMesh(core_axis_name="c", subcore_axis_name="s")
_SC_PARAMS = pltpu.CompilerParams(use_tc_tiling_on_sc=False)


def _make_agg(F):
    @functools.partial(
        pl.kernel,
        out_type=jax.ShapeDtypeStruct((2, NP, F), jnp.float32),
        mesh=_MESH,
        compiler_params=_SC_PARAMS,
        scratch_types=[
            pltpu.VMEM((NCH, CH), jnp.int32),
            pltpu.VMEM((NCH, CH), jnp.int32),
            pltpu.VMEM((NBC, CH, F), jnp.float32),
            pltpu.VMEM((NBC, CH, F), jnp.float32),
            pltpu.VMEM_SHARED((NP, F), jnp.float32),
            pltpu.SemaphoreType.DMA,
            pltpu.SemaphoreType.DMA,
            pltpu.SemaphoreType.DMA,
            pltpu.SemaphoreType.DMA,
        ],
    )
    def agg(g_hbm, ei_hbm, pad_hbm, z_hbm, out_hbm, src_v, dst_v, bufa, bufb,
            acc_sh, gsema, gsemb, ssema, ssemb):
        c = lax.axis_index("c")
        s = lax.axis_index("s")
        w = c * NS + s
        pltpu.sync_copy(z_hbm.at[pl.ds(s * RPT, RPT)],
                        acc_sh.at[pl.ds(s * RPT, RPT)])

        @pl.when(w < NW - 1)
        def _():
            pltpu.sync_copy(ei_hbm.at[0, pl.ds(w * NCH, NCH)], src_v)
            pltpu.sync_copy(ei_hbm.at[1, pl.ds(w * NCH, NCH)], dst_v)

        @pl.when(w == NW - 1)
        def _():
            pltpu.sync_copy(ei_hbm.at[0, pl.ds(w * NCH, NCHR)],
                            src_v.at[pl.ds(0, NCHR)])
            pltpu.sync_copy(ei_hbm.at[1, pl.ds(w * NCH, NCHR)],
                            dst_v.at[pl.ds(0, NCHR)])
            pltpu.sync_copy(pad_hbm.at[0], src_v.at[pl.ds(NCHR, NCH - NCHR)])
            pltpu.sync_copy(pad_hbm.at[1], dst_v.at[pl.ds(NCHR, NCH - NCHR)])

        plsc.subcore_barrier()

        def gather(ch, buf, b, sem):
            pltpu.async_copy(g_hbm.at[src_v.at[ch]], buf.at[b], sem)

        def gather_wait(buf, b, sem):
            pltpu.make_async_copy(g_hbm.at[src_v.at[0]], buf.at[b], sem).wait()

        def scat(ch, buf, b, sem):
            pltpu.async_copy(buf.at[b], acc_sh.at[dst_v.at[ch]], sem, add=True)

        def scat_wait(buf, b, sem):
            pltpu.make_async_copy(buf.at[b], acc_sh.at[dst_v.at[0]],
                                  sem).wait()

        for b in range(NBC):
            gather(b, bufa, b, gsema)

        def phase(j, buf, gsem, obuf, osem_s, gsem_o, first, last):
            for b in range(NBC):
                gather_wait(buf, b, gsem)
            for b in range(NBC):
                scat(j * NBC + b, buf, b,
                     ssema if buf is bufa else ssemb)
            if first is None:
                for b in range(NBC):
                    scat_wait(obuf, b, osem_s)
            else:
                @pl.when(jnp.logical_not(first))
                def _():
                    for b in range(NBC):
                        scat_wait(obuf, b, osem_s)

            @pl.when(jnp.logical_not(last))
            def _():
                for b in range(NBC):
                    gather((j + 1) * NBC + b, obuf, b, gsem_o)

        def body(j, carry):
            @pl.when(j % 2 == 0)
            def _():
                phase(j, bufa, gsema, bufb, ssemb, gsemb,
                      first=j == 0, last=j >= NG - 1)

            @pl.when(j % 2 == 1)
            def _():
                phase(j, bufb, gsemb, bufa, ssema, gsema,
                      first=None, last=j >= NG - 1)

            return carry

        lax.fori_loop(0, NG, body, 0)
        for b in range(NBC):
            scat_wait(bufb if NG % 2 == 0 else bufa, b,
                      ssemb if NG % 2 == 0 else ssema)
        plsc.subcore_barrier()
        pltpu.sync_copy(acc_sh.at[pl.ds(s * RPT, RPT)],
                        out_hbm.at[c, pl.ds(s * RPT, RPT)])

    return agg


_agg16 = _make_agg(16)


@functools.partial(
    pl.kernel,
    out_type=jax.ShapeDtypeStruct((2, NP, 16), jnp.float32),
    mesh=_MESH,
    compiler_params=_SC_PARAMS,
    scratch_types=[
        pltpu.VMEM((NCH, CH), jnp.int32),
        pltpu.VMEM((CH, 16), jnp.float32),
        pltpu.VMEM_SHARED((NP, 16), jnp.float32),
        pltpu.SemaphoreType.DMA,
    ],
)
def _deg(ei_hbm, pad_hbm, ones_hbm, z_hbm, out_hbm, dst_v, ones_v, acc_sh,
         sem):
    c = lax.axis_index("c")
    s = lax.axis_index("s")
    w = c * NS + s
    pltpu.sync_copy(z_hbm.at[pl.ds(s * RPT, RPT)],
                    acc_sh.at[pl.ds(s * RPT, RPT)])
    pltpu.sync_copy(ones_hbm, ones_v)

    @pl.when(w < NW - 1)
    def _():
        pltpu.sync_copy(ei_hbm.at[1, pl.ds(w * NCH, NCH)], dst_v)

    @pl.when(w == NW - 1)
    def _():
        pltpu.sync_copy(ei_hbm.at[1, pl.ds(w * NCH, NCHR)],
                        dst_v.at[pl.ds(0, NCHR)])
        pltpu.sync_copy(pad_hbm.at[1], dst_v.at[pl.ds(NCHR, NCH - NCHR)])

    plsc.subcore_barrier()

    K = 8

    def scat(ch):
        pltpu.async_copy(ones_v, acc_sh.at[dst_v.at[ch]], sem, add=True)

    def scat_wait():
        pltpu.make_async_copy(ones_v, acc_sh.at[dst_v.at[0]], sem).wait()

    for b in range(K):
        scat(b)

    def body(j, carry):
        for b in range(K):
            scat(j * K + b)
        for b in range(K):
            scat_wait()
        return carry

    lax.fori_loop(1, NCH // K, body, 0)
    for b in range(K):
        scat_wait()
    plsc.subcore_barrier()
    pltpu.sync_copy(acc_sh.at[pl.ds(s * RPT, RPT)],
                    out_hbm.at[c, pl.ds(s * RPT, RPT)])


NPF = NP * 16 // 128
NF1 = N_NODES * 16 // 128


def _k1a_body(x3_ref, we_ref, h_ref):
    hf = jnp.dot(x3_ref[:, 0, :], we_ref[0],
                 preferred_element_type=jnp.float32)
    for a in range(1, 8):
        hf = hf + jnp.dot(x3_ref[:, a, :], we_ref[a],
                          preferred_element_type=jnp.float32)
    h_ref[...] = hf


def _k1b_body(hf_ref, daccf_ref, g_ref, dis_ref):
    degf = daccf_ref[0] + daccf_ref[1] + 1.0
    disf = lax.rsqrt(degf)
    dis_ref[...] = disf
    g = hf_ref[...] * disf[:NF1]
    g_ref[...] = jnp.concatenate(
        [g, jnp.zeros((NPF - NF1, 128), jnp.float32)], axis=0)


def _k2_body(af_ref, gf_ref, disf_ref, bf_ref, wf_ref, out_ref):
    disf = disf_ref[...]
    z = jnp.maximum(disf * (af_ref[0] + af_ref[1] + gf_ref[...]) + bf_ref[...],
                    0.0)
    out_ref[...] = (
        jnp.dot(z, wf_ref[...], preferred_element_type=jnp.float32) * disf)


def _k3_body(af_ref, gf_ref, disf_ref, bf_ref, out_ref):
    disf = disf_ref[...]
    z = jnp.maximum(disf * (af_ref[0] + af_ref[1] + gf_ref[...]) + bf_ref[...],
                    0.0)
    out_ref[...] = z * disf


def _k4_body(af_ref, gf_ref, disf_ref, we_ref, b_ref, out_ref):
    qdf = disf_ref[...] * (af_ref[0] + af_ref[1] + gf_ref[...])
    cols = []
    for a in range(8):
        h = jnp.dot(qdf, we_ref[a], preferred_element_type=jnp.float32)
        p = h + b_ref[...]
        m = jnp.max(p, axis=1, keepdims=True)
        e = jnp.exp(p - m)
        o = p - m - jnp.log(jnp.sum(e, axis=1, keepdims=True))
        cols.append(jnp.reshape(o, (NPF, 1, 40)))
    full = jnp.concatenate(cols, axis=1).reshape(NP, 40)
    out_ref[...] = full[:N_NODES]


def _f32(*shape):
    return jax.ShapeDtypeStruct(shape, jnp.float32)


_K1A = pl.pallas_call(_k1a_body, out_shape=_f32(NF1, 128))
_K1B = pl.pallas_call(_k1b_body, out_shape=[_f32(NPF, 128), _f32(NPF, 128)])
_K2 = pl.pallas_call(_k2_body, out_shape=_f32(NPF, 128))
_K3 = pl.pallas_call(_k3_body, out_shape=_f32(NPF, 128))
_K4 = pl.pallas_call(_k4_body, out_shape=_f32(N_NODES, 40))


def kernel(x, edge_index, W1, b1, W2, b2, W3, b3):
    npad = E_PAD - N_EDGES
    k = jnp.arange(npad, dtype=jnp.int32)
    pad3 = jnp.stack([k % N_NODES, N_NODES + k % (NP - N_NODES)]
                     ).reshape(2, NCH - NCHR, CH)
    ei = edge_index.reshape(2, NCHT, CH)
    z16 = jnp.zeros((NP, 16), jnp.float32)
    ones = jnp.ones((CH, 16), jnp.float32)
    e8 = jnp.eye(8, dtype=jnp.float32)
    w1e = (e8[:, None, :, None] * W1[None, :, None, :]).reshape(8, 128, 128)
    w2f = (e8[:, None, :, None] * W2[None, :, None, :]).reshape(128, 128)

    hf = _K1A(x.reshape(NF1, 8, 128), w1e)
    dacc = _deg(ei, pad3, ones, z16)
    g1f, disf = _K1B(hf, dacc.reshape(2, NPF, 128))
    a1 = _agg16(g1f.reshape(NP, 16), ei, pad3, z16)
    g2f = _K2(a1.reshape(2, NPF, 128), g1f, disf,
              jnp.tile(b1, 8).reshape(1, 128), w2f)
    a2 = _agg16(g2f.reshape(NP, 16), ei, pad3, z16)
    g3f = _K3(a2.reshape(2, NPF, 128), g2f, disf,
              jnp.tile(b2, 8).reshape(1, 128))
    a3 = _agg16(g3f.reshape(NP, 16), ei, pad3, z16)
    w3e = (e8[:, :, None, None] * W3[None, None, :, :]).reshape(8, 128, 40)
    return _K4(a3.reshape(2, NPF, 128), g3f, disf, w3e, b3.reshape(1, 40))
